# baseline (speedup 1.0000x reference)
import functools

import jax
import jax.numpy as jnp
from jax import lax
from jax.experimental import pallas as pl
from jax.experimental.pallas import tpu as pltpu
from jax.experimental.pallas import tpu_sc as plsc

L = 16
NC, NS = 2, 16
NW = NC * NS
B, D = 128, 32768
SC_ROWS = 32
TC_ROWS = B - SC_ROWS
CHUNK = 16384
CHUNKS_PER_ROW = D // CHUNK
NCHUNK = CHUNKS_PER_ROW
UNROLL = 8
BM, BN = 32, 8192


def _orderable_key(x, pos):
    rpos = (~pos) & 3
    s = lax.bitcast_convert_type(x, jnp.int32)
    ordv = s ^ (lax.shift_right_arithmetic(s, 31) & jnp.int32(0x7FFFFFFF))
    return (ordv & jnp.int32(~3)) | rpos


def _shuffle(x, idx):
    return lax.gather(
        x,
        idx[:, None],
        lax.GatherDimensionNumbers(
            offset_dims=(), collapsed_slice_dims=(0,), start_index_map=(0,)
        ),
        slice_sizes=(1,),
        mode=lax.GatherScatterMode.PROMISE_IN_BOUNDS,
    )


def _lwta_vreg(x):
    iota = lax.iota(jnp.int32, L)
    key = _orderable_key(x, iota & 3)
    km = jnp.maximum(key, _shuffle(key, iota ^ 1))
    km = jnp.maximum(km, _shuffle(km, iota ^ 2))
    return jnp.where(key == km, x, 0.0)


def _compute_chunk(in_v, out_v):
    def body(j, _):
        o = j * (UNROLL * L)
        for k in range(UNROLL):
            s = pl.ds(o + k * L, L)
            out_v[s] = _lwta_vreg(in_v[s])
        return 0

    lax.fori_loop(0, CHUNK // (UNROLL * L), body, 0)


@functools.partial(
    pl.kernel,
    mesh=plsc.VectorSubcoreMesh(core_axis_name="c", subcore_axis_name="s"),
    out_type=jax.ShapeDtypeStruct((SC_ROWS, D), jnp.float32),
    scratch_types=[
        pltpu.VMEM((CHUNK,), jnp.float32),
        pltpu.VMEM((CHUNK,), jnp.float32),
        pltpu.VMEM((CHUNK,), jnp.float32),
        pltpu.VMEM((CHUNK,), jnp.float32),
        pltpu.SemaphoreType.DMA,
        pltpu.SemaphoreType.DMA,
        pltpu.SemaphoreType.DMA,
        pltpu.SemaphoreType.DMA,
    ],
)
def _lwta_sc(x_hbm, o_hbm, in0, in1, out0, out1, s_in0, s_in1, s_out0, s_out1):
    wid = lax.axis_index("s") * NC + lax.axis_index("c")
    ins, outs = [in0, in1], [out0, out1]
    s_ins, s_outs = [s_in0, s_in1], [s_out0, s_out1]
    in_h = [None] * NCHUNK
    out_h = [None] * NCHUNK

    def src(c):
        return x_hbm.at[TC_ROWS + wid, pl.ds(c * CHUNK, CHUNK)]

    def dst(c):
        return o_hbm.at[wid, pl.ds(c * CHUNK, CHUNK)]

    in_h[0] = pltpu.async_copy(src(0), ins[0], s_ins[0])
    for c in range(NCHUNK):
        b = c % 2
        if c + 1 < NCHUNK:
            nb = (c + 1) % 2
            in_h[c + 1] = pltpu.async_copy(src(c + 1), ins[nb], s_ins[nb])
        in_h[c].wait()
        if c >= 2:
            out_h[c - 2].wait()
        _compute_chunk(ins[b], outs[b])
        out_h[c] = pltpu.async_copy(outs[b], dst(c), s_outs[b])
    out_h[NCHUNK - 2].wait()
    out_h[NCHUNK - 1].wait()


def _tc_body(x_ref, o_ref):
    x = x_ref[...]
    pos = lax.broadcasted_iota(jnp.int32, x.shape, 1) & 3
    even = (pos & 1) == 0
    low = (pos & 2) == 0
    key = _orderable_key(x, pos)
    p1 = jnp.where(even, pltpu.roll(key, BN - 1, 1), pltpu.roll(key, 1, 1))
    km = jnp.maximum(key, p1)
    p2 = jnp.where(low, pltpu.roll(km, BN - 2, 1), pltpu.roll(km, 2, 1))
    km = jnp.maximum(km, p2)
    o_ref[...] = jnp.where(key == km, x, 0.0)


_lwta_tc = pl.pallas_call(
    _tc_body,
    grid=(TC_ROWS // BM, D // BN),
    in_specs=[pl.BlockSpec((BM, BN), lambda i, j: (i, j))],
    out_specs=pl.BlockSpec((BM, BN), lambda i, j: (i, j)),
    out_shape=jax.ShapeDtypeStruct((TC_ROWS, D), jnp.float32),
)


def kernel(inputs):
    sc_part = _lwta_sc(inputs)
    tc_part = _lwta_tc(inputs[:TC_ROWS])
    return jnp.concatenate([tc_part, sc_part], axis=0)


# speedup vs baseline: 1.1976x; 1.1976x over previous
import functools

import jax
import jax.numpy as jnp
from jax import lax
from jax.experimental import pallas as pl
from jax.experimental.pallas import tpu as pltpu
from jax.experimental.pallas import tpu_sc as plsc

L = 16
NC, NS = 2, 16
NW = NC * NS
B, D = 128, 32768
SC_ROWS = 32
TC_ROWS = B - SC_ROWS
CHUNK = 16384
CHUNKS_PER_ROW = D // CHUNK
NCHUNK = CHUNKS_PER_ROW
UNROLL = 8
BM, BN = 32, 8192


def _orderable_key(x, pos):
    rpos = (~pos) & 3
    s = lax.bitcast_convert_type(x, jnp.int32)
    ordv = s ^ (lax.shift_right_arithmetic(s, 31) & jnp.int32(0x7FFFFFFF))
    return (ordv & jnp.int32(~3)) | rpos


def _shuffle(x, idx):
    return lax.gather(
        x,
        idx[:, None],
        lax.GatherDimensionNumbers(
            offset_dims=(), collapsed_slice_dims=(0,), start_index_map=(0,)
        ),
        slice_sizes=(1,),
        mode=lax.GatherScatterMode.PROMISE_IN_BOUNDS,
    )


def _lwta_vreg(x):
    iota = lax.iota(jnp.int32, L)
    key = _orderable_key(x, iota & 3)
    km = jnp.maximum(key, _shuffle(key, iota ^ 1))
    km = jnp.maximum(km, _shuffle(km, iota ^ 2))
    return jnp.where(key == km, x, 0.0)


def _compute_chunk(in_v, out_v):
    def body(j, _):
        o = j * (UNROLL * L)
        for k in range(UNROLL):
            s = pl.ds(o + k * L, L)
            out_v[s] = _lwta_vreg(in_v[s])
        return 0

    lax.fori_loop(0, CHUNK // (UNROLL * L), body, 0)


@functools.partial(
    pl.kernel,
    mesh=plsc.VectorSubcoreMesh(core_axis_name="c", subcore_axis_name="s"),
    out_type=jax.ShapeDtypeStruct((SC_ROWS, D), jnp.float32),
    scratch_types=[
        pltpu.VMEM((CHUNK,), jnp.float32),
        pltpu.VMEM((CHUNK,), jnp.float32),
        pltpu.VMEM((CHUNK,), jnp.float32),
        pltpu.VMEM((CHUNK,), jnp.float32),
        pltpu.SemaphoreType.DMA,
        pltpu.SemaphoreType.DMA,
        pltpu.SemaphoreType.DMA,
        pltpu.SemaphoreType.DMA,
    ],
)
def _lwta_sc(x_hbm, o_hbm, in0, in1, out0, out1, s_in0, s_in1, s_out0, s_out1):
    wid = lax.axis_index("s") * NC + lax.axis_index("c")
    ins, outs = [in0, in1], [out0, out1]
    s_ins, s_outs = [s_in0, s_in1], [s_out0, s_out1]
    in_h = [None] * NCHUNK
    out_h = [None] * NCHUNK

    def src(c):
        return x_hbm.at[TC_ROWS + wid, pl.ds(c * CHUNK, CHUNK)]

    def dst(c):
        return o_hbm.at[wid, pl.ds(c * CHUNK, CHUNK)]

    in_h[0] = pltpu.async_copy(src(0), ins[0], s_ins[0])
    for c in range(NCHUNK):
        b = c % 2
        if c + 1 < NCHUNK:
            nb = (c + 1) % 2
            in_h[c + 1] = pltpu.async_copy(src(c + 1), ins[nb], s_ins[nb])
        in_h[c].wait()
        if c >= 2:
            out_h[c - 2].wait()
        _compute_chunk(ins[b], outs[b])
        out_h[c] = pltpu.async_copy(outs[b], dst(c), s_outs[b])
    out_h[NCHUNK - 2].wait()
    out_h[NCHUNK - 1].wait()


def _tc_body(x_ref, o_ref):
    x = x_ref[...]
    pos = lax.broadcasted_iota(jnp.int32, x.shape, 1) & 3
    even = (pos & 1) == 0
    low = (pos & 2) == 0
    key = _orderable_key(x, pos)
    p1 = jnp.where(even, pltpu.roll(key, BN - 1, 1), pltpu.roll(key, 1, 1))
    km = jnp.maximum(key, p1)
    p2 = jnp.where(low, pltpu.roll(km, BN - 2, 1), pltpu.roll(km, 2, 1))
    km = jnp.maximum(km, p2)
    o_ref[...] = jnp.where(key == km, x, 0.0)


_lwta_tc = pl.pallas_call(
    _tc_body,
    grid=(TC_ROWS // BM, D // BN),
    in_specs=[pl.BlockSpec((BM, BN), lambda i, j: (i, j))],
    out_specs=pl.BlockSpec((BM, BN), lambda i, j: (i, j)),
    out_shape=jax.ShapeDtypeStruct((TC_ROWS, D), jnp.float32),
)


def kernel(inputs):
    sc_part = _lwta_sc(inputs)
    tc_part = _lwta_tc(inputs)
    return jnp.concatenate([tc_part, sc_part], axis=0)


# speedup vs baseline: 1.2371x; 1.0330x over previous
import functools

import jax
import jax.numpy as jnp
from jax import lax
from jax.experimental import pallas as pl
from jax.experimental.pallas import tpu as pltpu
from jax.experimental.pallas import tpu_sc as plsc

L = 16
NC, NS = 2, 16
NW = NC * NS
B, D = 128, 32768
SC_ROWS = 32
TC_ROWS = B - SC_ROWS
CHUNK = 16384
CHUNKS_PER_ROW = D // CHUNK
NCHUNK = CHUNKS_PER_ROW
UNROLL = 8
BM, BN = 32, 8192


def _orderable_key(x, pos):
    rpos = (~pos) & 3
    s = lax.bitcast_convert_type(x, jnp.int32)
    ordv = s ^ (lax.shift_right_arithmetic(s, 31) & jnp.int32(0x7FFFFFFF))
    return (ordv & jnp.int32(~3)) | rpos


def _shuffle(x, idx):
    return lax.gather(
        x,
        idx[:, None],
        lax.GatherDimensionNumbers(
            offset_dims=(), collapsed_slice_dims=(0,), start_index_map=(0,)
        ),
        slice_sizes=(1,),
        mode=lax.GatherScatterMode.PROMISE_IN_BOUNDS,
    )


def _lwta_vreg(x):
    iota = lax.iota(jnp.int32, L)
    key = _orderable_key(x, iota & 3)
    km = jnp.maximum(key, _shuffle(key, iota ^ 1))
    km = jnp.maximum(km, _shuffle(km, iota ^ 2))
    return jnp.where(key == km, x, 0.0)


def _compute_chunk(in_v, out_v):
    def body(j, _):
        o = j * (UNROLL * L)
        for k in range(UNROLL):
            s = pl.ds(o + k * L, L)
            out_v[s] = _lwta_vreg(in_v[s])
        return 0

    lax.fori_loop(0, CHUNK // (UNROLL * L), body, 0)


@functools.partial(
    pl.kernel,
    mesh=plsc.VectorSubcoreMesh(core_axis_name="c", subcore_axis_name="s"),
    out_type=jax.ShapeDtypeStruct((B, D), jnp.float32),
    scratch_types=[
        pltpu.VMEM((CHUNK,), jnp.float32),
        pltpu.VMEM((CHUNK,), jnp.float32),
        pltpu.VMEM((CHUNK,), jnp.float32),
        pltpu.VMEM((CHUNK,), jnp.float32),
        pltpu.SemaphoreType.DMA,
        pltpu.SemaphoreType.DMA,
        pltpu.SemaphoreType.DMA,
        pltpu.SemaphoreType.DMA,
    ],
)
def _lwta_sc(x_hbm, o_hbm, in0, in1, out0, out1, s_in0, s_in1, s_out0, s_out1):
    wid = lax.axis_index("s") * NC + lax.axis_index("c")
    ins, outs = [in0, in1], [out0, out1]
    s_ins, s_outs = [s_in0, s_in1], [s_out0, s_out1]
    in_h = [None] * NCHUNK
    out_h = [None] * NCHUNK

    def src(c):
        return x_hbm.at[TC_ROWS + wid, pl.ds(c * CHUNK, CHUNK)]

    def dst(c):
        return o_hbm.at[TC_ROWS + wid, pl.ds(c * CHUNK, CHUNK)]

    in_h[0] = pltpu.async_copy(src(0), ins[0], s_ins[0])
    for c in range(NCHUNK):
        b = c % 2
        if c + 1 < NCHUNK:
            nb = (c + 1) % 2
            in_h[c + 1] = pltpu.async_copy(src(c + 1), ins[nb], s_ins[nb])
        in_h[c].wait()
        if c >= 2:
            out_h[c - 2].wait()
        _compute_chunk(ins[b], outs[b])
        out_h[c] = pltpu.async_copy(outs[b], dst(c), s_outs[b])
    out_h[NCHUNK - 2].wait()
    out_h[NCHUNK - 1].wait()


def _tc_body(x_ref, alias_ref, o_ref):
    x = x_ref[...]
    pos = lax.broadcasted_iota(jnp.int32, x.shape, 1) & 3
    even = (pos & 1) == 0
    low = (pos & 2) == 0
    key = _orderable_key(x, pos)
    p1 = jnp.where(even, pltpu.roll(key, BN - 1, 1), pltpu.roll(key, 1, 1))
    km = jnp.maximum(key, p1)
    p2 = jnp.where(low, pltpu.roll(km, BN - 2, 1), pltpu.roll(km, 2, 1))
    km = jnp.maximum(km, p2)
    o_ref[...] = jnp.where(key == km, x, 0.0)


_lwta_tc = pl.pallas_call(
    _tc_body,
    grid=(TC_ROWS // BM, D // BN),
    in_specs=[
        pl.BlockSpec((BM, BN), lambda i, j: (i, j)),
        pl.BlockSpec(memory_space=pl.ANY),
    ],
    out_specs=pl.BlockSpec((BM, BN), lambda i, j: (i, j)),
    out_shape=jax.ShapeDtypeStruct((B, D), jnp.float32),
    input_output_aliases={1: 0},
)


def kernel(inputs):
    sc_full = _lwta_sc(inputs)
    return _lwta_tc(inputs, sc_full)


# speedup vs baseline: 1.3176x; 1.0651x over previous
import functools

import jax
import jax.numpy as jnp
from jax import lax
from jax.experimental import pallas as pl
from jax.experimental.pallas import tpu as pltpu
from jax.experimental.pallas import tpu_sc as plsc

L = 16
NC, NS = 2, 16
NW = NC * NS
B, D = 128, 32768
ROWS_PER_W = B // NW
CHUNK = 16384
CHUNKS_PER_ROW = D // CHUNK
NCHUNK = ROWS_PER_W * CHUNKS_PER_ROW
UNROLL = 8


def _shuffle(x, idx):
    return lax.gather(
        x,
        idx[:, None],
        lax.GatherDimensionNumbers(
            offset_dims=(), collapsed_slice_dims=(0,), start_index_map=(0,)
        ),
        slice_sizes=(1,),
        mode=lax.GatherScatterMode.PROMISE_IN_BOUNDS,
    )


def _lwta_vreg(x):
    iota = lax.iota(jnp.int32, L)
    i1 = iota ^ 1
    i2 = iota ^ 2
    rpos = (~iota) & 3
    s = lax.bitcast_convert_type(x, jnp.int32)
    ordv = s ^ (lax.shift_right_arithmetic(s, 31) & jnp.int32(0x7FFFFFFF))
    key = (ordv & jnp.int32(~3)) | rpos
    km = jnp.maximum(key, _shuffle(key, i1))
    km = jnp.maximum(km, _shuffle(km, i2))
    return jnp.where(key == km, x, 0.0)


def _compute_chunk(in_v, out_v):
    def body(j, _):
        o = j * (UNROLL * L)
        for k in range(UNROLL):
            s = pl.ds(o + k * L, L)
            out_v[s] = _lwta_vreg(in_v[s])
        return 0

    lax.fori_loop(0, CHUNK // (UNROLL * L), body, 0)


@functools.partial(
    pl.kernel,
    mesh=plsc.VectorSubcoreMesh(core_axis_name="c", subcore_axis_name="s"),
    compiler_params=pltpu.CompilerParams(skip_device_barrier=True),
    out_type=jax.ShapeDtypeStruct((B, D), jnp.float32),
    scratch_types=[
        pltpu.VMEM((CHUNK,), jnp.float32),
        pltpu.VMEM((CHUNK,), jnp.float32),
        pltpu.VMEM((CHUNK,), jnp.float32),
        pltpu.VMEM((CHUNK,), jnp.float32),
        pltpu.SemaphoreType.DMA,
        pltpu.SemaphoreType.DMA,
        pltpu.SemaphoreType.DMA,
        pltpu.SemaphoreType.DMA,
    ],
)
def _lwta_sc(x_hbm, o_hbm, in0, in1, out0, out1, s_in0, s_in1, s_out0, s_out1):
    wid = lax.axis_index("s") * NC + lax.axis_index("c")
    row0 = wid * ROWS_PER_W
    ins, outs = [in0, in1], [out0, out1]
    s_ins, s_outs = [s_in0, s_in1], [s_out0, s_out1]
    in_h = [None] * NCHUNK
    out_h = [None] * NCHUNK

    def src(c):
        return x_hbm.at[row0 + c // CHUNKS_PER_ROW,
                        pl.ds((c % CHUNKS_PER_ROW) * CHUNK, CHUNK)]

    def dst(c):
        return o_hbm.at[row0 + c // CHUNKS_PER_ROW,
                        pl.ds((c % CHUNKS_PER_ROW) * CHUNK, CHUNK)]

    in_h[0] = pltpu.async_copy(src(0), ins[0], s_ins[0])
    for c in range(NCHUNK):
        b = c % 2
        if c + 1 < NCHUNK:
            nb = (c + 1) % 2
            in_h[c + 1] = pltpu.async_copy(src(c + 1), ins[nb], s_ins[nb])
        in_h[c].wait()
        if c >= 2:
            out_h[c - 2].wait()
        _compute_chunk(ins[b], outs[b])
        out_h[c] = pltpu.async_copy(outs[b], dst(c), s_outs[b])
    out_h[NCHUNK - 2].wait()
    out_h[NCHUNK - 1].wait()


def kernel(inputs):
    return _lwta_sc(inputs)
